# Initial kernel scaffold; baseline (speedup 1.0000x reference)
import jax
import jax.numpy as jnp
from jax.experimental import pallas as pl


def kernel(volume, gt_mask, gt_skel):
    raise NotImplementedError("write your pallas kernel here")


# speedup vs baseline: 4.9217x; 4.9217x over previous
import functools

import jax
import jax.numpy as jnp
import numpy as np
from jax import lax
from jax.experimental import pallas as pl
from jax.experimental.pallas import tpu as pltpu
from jax.experimental.pallas import tpu_sc as plsc

_SZ = 96
_B, _D, _H, _W = 4, 128, 128, 128

_BZ = (28, 12, 5, 26)
_BY = (20, 17, 22, 23)
_BX = (4, 21, 4, 15)

_NC = 2
_NS = 16
_NW = _NC * _NS
_ZPW = _SZ // _NW

_IN_CHUNK = _SZ * _W
_OUT_CHUNK = _SZ * _SZ
_OUT_PLANE = _SZ * _SZ
_OUT_BATCH = _SZ * _SZ * _SZ


def _crop_body(vol, msk, skl, o0, o1, o2,
               ib0, ib1, ob0, ob1, si0, si1, so0, so1):
    ins = (vol, msk, skl)
    outs = (o0, o1, o2)
    ibufs = (ib0, ib1)
    obufs = (ob0, ob1)
    isems = (si0, si1)
    osems = (so0, so1)

    wid = lax.axis_index("s") * _NC + lax.axis_index("c")
    iota = lax.iota(jnp.int32, 16)

    idx_vecs = [[iota + (_BX[b] + 16 * k) for k in range(6)] for b in range(_B)]

    chunks = [(b, t, j) for b in range(_B) for t in range(3) for j in range(_ZPW)]

    def in_start(b, j):
        z = wid * _ZPW + j
        base = ((b * _D + _BZ[b]) * _H + _BY[b]) * _W
        return pl.multiple_of(base + z * _H * _W, _W)

    def out_start(b, j):
        z = wid * _ZPW + j
        return pl.multiple_of(b * _OUT_BATCH + z * _OUT_PLANE, _OUT_PLANE)

    def issue_in(i):
        b, t, j = chunks[i]
        return pltpu.async_copy(
            ins[t].at[pl.ds(in_start(b, j), _IN_CHUNK)], ibufs[i % 2], isems[i % 2])

    def issue_out(i):
        b, t, j = chunks[i]
        return pltpu.async_copy(
            obufs[i % 2], outs[t].at[pl.ds(out_start(b, j), _OUT_CHUNK)], osems[i % 2])

    def repack(i):
        b, _, _ = chunks[i]
        ib = ibufs[i % 2]
        ob = obufs[i % 2]
        vecs = idx_vecs[b]

        def row(y, carry):
            rbase = y * _W + _BX[b]
            obase = y * _SZ
            for k in range(6):
                v = ib[pl.ds(rbase + 16 * k, 16)]
                ob[pl.ds(obase + 16 * k, 16)] = v
            return carry

        lax.fori_loop(0, _SZ, row, 0)

    n = len(chunks)
    out_handles = [None] * n
    pending_in = issue_in(0)
    for i in range(n):
        next_in = issue_in(i + 1) if i + 1 < n else None
        pending_in.wait()
        if i >= 2:
            out_handles[i - 2].wait()
        repack(i)
        out_handles[i] = issue_out(i)
        pending_in = next_in
    out_handles[n - 2].wait()
    out_handles[n - 1].wait()


@jax.jit
def _crop_call(vol, msk, skl):
    f32 = jnp.float32
    out_sds = jax.ShapeDtypeStruct((_B * _OUT_BATCH,), f32)
    run = pl.kernel(
        _crop_body,
        out_type=[out_sds, out_sds, out_sds],
        mesh=plsc.VectorSubcoreMesh(core_axis_name="c", subcore_axis_name="s"),
        scratch_types=[
            pltpu.VMEM((_IN_CHUNK,), f32),
            pltpu.VMEM((_IN_CHUNK,), f32),
            pltpu.VMEM((_OUT_CHUNK,), f32),
            pltpu.VMEM((_OUT_CHUNK,), f32),
            pltpu.SemaphoreType.DMA,
            pltpu.SemaphoreType.DMA,
            pltpu.SemaphoreType.DMA,
            pltpu.SemaphoreType.DMA,
        ],
    )
    return run(vol.reshape(-1), msk.reshape(-1), skl.reshape(-1))


def kernel(volume, gt_mask, gt_skel):
    o0, o1, o2 = _crop_call(volume, gt_mask, gt_skel)
    shape = (_B, _SZ, _SZ, _SZ)
    return (o0.reshape(shape), o1.reshape(shape), o2.reshape(shape))


# speedup vs baseline: 12.0883x; 2.4561x over previous
import jax
import jax.numpy as jnp
from jax import lax
from jax.experimental import pallas as pl
from jax.experimental.pallas import tpu as pltpu
from jax.experimental.pallas import tpu_sc as plsc

_SZ = 96
_B, _D, _H, _W = 4, 128, 128, 128

_BZ = (28, 12, 5, 26)
_BY = (20, 17, 22, 23)
_BX = (4, 21, 4, 15)

_NC = 2
_NS = 16
_NW = _NC * _NS
_ZPW = _SZ // _NW

_CHUNK = _SZ * _W
_NBUF = 3


def _crop_body(vol, msk, skl, o0, o1, o2,
               b0, b1, b2, si0, si1, si2, so0, so1, so2):
    ins = (vol, msk, skl)
    outs = (o0, o1, o2)
    bufs = (b0, b1, b2)
    isems = (si0, si1, si2)
    osems = (so0, so1, so2)

    wid = lax.axis_index("s") * _NC + lax.axis_index("c")

    chunks = [(b, t, j) for b in range(_B) for t in range(3) for j in range(_ZPW)]

    def in_start(b, j):
        z = wid * _ZPW + j
        return pl.multiple_of(((b * _D + _BZ[b] + z) * _H + _BY[b]) * _W, _W)

    def out_start(b, j):
        z = wid * _ZPW + j
        return pl.multiple_of((b * _SZ + z) * _CHUNK, _CHUNK)

    def issue_in(i):
        b, t, j = chunks[i]
        return pltpu.async_copy(
            ins[t].at[pl.ds(in_start(b, j), _CHUNK)], bufs[i % _NBUF],
            isems[i % _NBUF])

    def issue_out(i):
        b, t, j = chunks[i]
        return pltpu.async_copy(
            bufs[i % _NBUF], outs[t].at[pl.ds(out_start(b, j), _CHUNK)],
            osems[i % _NBUF])

    def repack(i):
        b, _, _ = chunks[i]
        bx = _BX[b]
        buf = bufs[i % _NBUF]

        def rows(y4, carry):
            for r in range(4):
                rbase = (y4 * 4 + r) * _W
                for k in range(6):
                    v = buf[pl.ds(rbase + bx + 16 * k, 16)]
                    buf[pl.ds(rbase + 16 * k, 16)] = v
            return carry

        lax.fori_loop(0, _SZ // 4, rows, 0)

    n = len(chunks)
    out_handles = [None] * n
    in_handles = [None] * n
    for i in range(_NBUF):
        in_handles[i] = issue_in(i)
    for i in range(n):
        in_handles[i].wait()
        repack(i)
        out_handles[i] = issue_out(i)
        if i + _NBUF < n:
            out_handles[i].wait()
            in_handles[i + _NBUF] = issue_in(i + _NBUF)
    for i in range(n - _NBUF, n):
        out_handles[i].wait()


@jax.jit
def _crop_call(vol, msk, skl):
    f32 = jnp.float32
    out_sds = jax.ShapeDtypeStruct((_B * _SZ * _SZ * _W,), f32)
    run = pl.kernel(
        _crop_body,
        out_type=[out_sds, out_sds, out_sds],
        mesh=plsc.VectorSubcoreMesh(core_axis_name="c", subcore_axis_name="s"),
        scratch_types=[
            pltpu.VMEM((_CHUNK,), f32),
            pltpu.VMEM((_CHUNK,), f32),
            pltpu.VMEM((_CHUNK,), f32),
            pltpu.SemaphoreType.DMA,
            pltpu.SemaphoreType.DMA,
            pltpu.SemaphoreType.DMA,
            pltpu.SemaphoreType.DMA,
            pltpu.SemaphoreType.DMA,
            pltpu.SemaphoreType.DMA,
        ],
    )
    return run(vol.reshape(-1), msk.reshape(-1), skl.reshape(-1))


def kernel(volume, gt_mask, gt_skel):
    o0, o1, o2 = _crop_call(volume, gt_mask, gt_skel)
    shape = (_B, _SZ, _SZ, _SZ)

    def depad(o):
        return o.reshape(_B * _SZ * _SZ, _W)[:, :_SZ].reshape(shape)

    return (depad(o0), depad(o1), depad(o2))
